# baseline (speedup 1.0000x reference)
import jax
import jax.numpy as jnp
from jax.experimental import pallas as pl
from jax.experimental.pallas import tpu as pltpu

_BLOCK = 2000


def _mpn_block_kernel(x1_ref, x2_ref, x3_ref, w1_ref, w2_ref, w3_ref, b_ref,
                      out_ref):
    acc = jnp.dot(x1_ref[...], w1_ref[...], preferred_element_type=jnp.float32)
    acc = acc + jnp.dot(x3_ref[...], w3_ref[...],
                        preferred_element_type=jnp.float32)
    y2 = jnp.dot(x2_ref[...], w2_ref[...], preferred_element_type=jnp.float32)
    acc = acc + y2.reshape(acc.shape)
    acc = acc + b_ref[...]
    out_ref[...] = jnp.maximum(acc, 0.0)


def kernel(f_src, f, sum_msg, w1, w2, w3, b):
    e, d_ndata = f_src.shape
    d_edata = f.shape[1]
    d_msg = sum_msg.shape[1]

    p1 = 128 // d_ndata
    p2 = 128 // d_edata
    p3 = 128 // d_msg

    x1 = f_src.reshape(e // p1, 128)
    x2 = f.reshape(e // p2, 128)
    x3 = sum_msg.reshape(e // p3, 128)
    w1b = jnp.kron(jnp.eye(p1, dtype=jnp.float32), w1)
    w2b = jnp.kron(jnp.eye(p2, dtype=jnp.float32), w2)
    w3b = jnp.kron(jnp.eye(p3, dtype=jnp.float32), w3)
    bt = jnp.tile(b, (1, p3))

    ep = e // p3
    block = _BLOCK if ep % _BLOCK == 0 else ep
    grid = ep // block
    bf = block // 2

    out = pl.pallas_call(
        _mpn_block_kernel,
        grid=(grid,),
        in_specs=[
            pl.BlockSpec((block, 128), lambda i: (i, 0)),
            pl.BlockSpec((bf, 128), lambda i: (i, 0)),
            pl.BlockSpec((block, 128), lambda i: (i, 0)),
            pl.BlockSpec((128, 128), lambda i: (0, 0)),
            pl.BlockSpec((128, p2 * d_msg), lambda i: (0, 0)),
            pl.BlockSpec((128, 128), lambda i: (0, 0)),
            pl.BlockSpec((1, 128), lambda i: (0, 0)),
        ],
        out_specs=pl.BlockSpec((block, 128), lambda i: (i, 0)),
        out_shape=jax.ShapeDtypeStruct((ep, 128), jnp.float32),
        compiler_params=pltpu.CompilerParams(
            dimension_semantics=("parallel",)),
    )(x1, x2, x3, w1b, w2b, w3b, bt)
    return out.reshape(e, d_msg)


# speedup vs baseline: 1.0245x; 1.0245x over previous
import jax
import jax.numpy as jnp
from jax.experimental import pallas as pl
from jax.experimental.pallas import tpu as pltpu

_BLOCK = 8000


def _mpn_block_kernel(x1_ref, x2_ref, x3_ref, w1_ref, w2_ref, w3_ref, b_ref,
                      out_ref):
    acc = jnp.dot(x1_ref[...], w1_ref[...], preferred_element_type=jnp.float32)
    acc = acc + jnp.dot(x3_ref[...], w3_ref[...],
                        preferred_element_type=jnp.float32)
    y2 = jnp.dot(x2_ref[...], w2_ref[...], preferred_element_type=jnp.float32)
    acc = acc + y2.reshape(acc.shape)
    acc = acc + b_ref[...]
    out_ref[...] = jnp.maximum(acc, 0.0)


def kernel(f_src, f, sum_msg, w1, w2, w3, b):
    e, d_ndata = f_src.shape
    d_edata = f.shape[1]
    d_msg = sum_msg.shape[1]

    p1 = 128 // d_ndata
    p2 = 128 // d_edata
    p3 = 128 // d_msg

    x1 = f_src.reshape(e // p1, 128)
    x2 = f.reshape(e // p2, 128)
    x3 = sum_msg.reshape(e // p3, 128)
    w1b = jnp.kron(jnp.eye(p1, dtype=jnp.float32), w1)
    w2b = jnp.kron(jnp.eye(p2, dtype=jnp.float32), w2)
    w3b = jnp.kron(jnp.eye(p3, dtype=jnp.float32), w3)
    bt = jnp.tile(b, (1, p3))

    ep = e // p3
    block = _BLOCK if ep % _BLOCK == 0 else ep
    grid = ep // block
    bf = block // 2

    out = pl.pallas_call(
        _mpn_block_kernel,
        grid=(grid,),
        in_specs=[
            pl.BlockSpec((block, 128), lambda i: (i, 0)),
            pl.BlockSpec((bf, 128), lambda i: (i, 0)),
            pl.BlockSpec((block, 128), lambda i: (i, 0)),
            pl.BlockSpec((128, 128), lambda i: (0, 0)),
            pl.BlockSpec((128, p2 * d_msg), lambda i: (0, 0)),
            pl.BlockSpec((128, 128), lambda i: (0, 0)),
            pl.BlockSpec((1, 128), lambda i: (0, 0)),
        ],
        out_specs=pl.BlockSpec((block, 128), lambda i: (i, 0)),
        out_shape=jax.ShapeDtypeStruct((ep, 128), jnp.float32),
        compiler_params=pltpu.CompilerParams(
            dimension_semantics=("parallel",)),
    )(x1, x2, x3, w1b, w2b, w3b, bt)
    return out.reshape(e, d_msg)


# speedup vs baseline: 12.0179x; 11.7301x over previous
import jax
import jax.numpy as jnp
from jax.experimental import pallas as pl
from jax.experimental.pallas import tpu as pltpu

_BN = 12800


def _mpn_block_kernel(x1_ref, x2_ref, x3_ref, w1_ref, w2_ref, w3_ref, b_ref,
                      out_ref):
    dn = (((0,), (0,)), ((), ()))
    a1 = jax.lax.dot_general(w1_ref[...], x1_ref[...], dn,
                             preferred_element_type=jnp.float32)
    a2 = jax.lax.dot_general(w2_ref[...], x2_ref[...], dn,
                             preferred_element_type=jnp.float32)
    a3 = jax.lax.dot_general(w3_ref[...], x3_ref[...], dn,
                             preferred_element_type=jnp.float32)
    acc = a1 + a2 + a3 + b_ref[...][:, 0:1]
    out_ref[...] = jnp.maximum(acc, 0.0)


def kernel(f_src, f, sum_msg, w1, w2, w3, b):
    e, d_ndata = f_src.shape
    d_edata = f.shape[1]
    d_msg = sum_msg.shape[1]

    x1 = f_src.T
    x2 = f.T
    x3 = sum_msg.T
    bt = jnp.tile(b.reshape(d_msg, 1), (1, 128))

    bn = _BN if e % _BN == 0 else e
    grid = e // bn

    out = pl.pallas_call(
        _mpn_block_kernel,
        grid=(grid,),
        in_specs=[
            pl.BlockSpec((d_ndata, bn), lambda i: (0, i)),
            pl.BlockSpec((d_edata, bn), lambda i: (0, i)),
            pl.BlockSpec((d_msg, bn), lambda i: (0, i)),
            pl.BlockSpec((d_ndata, d_msg), lambda i: (0, 0)),
            pl.BlockSpec((d_edata, d_msg), lambda i: (0, 0)),
            pl.BlockSpec((d_msg, d_msg), lambda i: (0, 0)),
            pl.BlockSpec((d_msg, 128), lambda i: (0, 0)),
        ],
        out_specs=pl.BlockSpec((d_msg, bn), lambda i: (0, i)),
        out_shape=jax.ShapeDtypeStruct((d_msg, e), jnp.float32),
        compiler_params=pltpu.CompilerParams(
            dimension_semantics=("parallel",)),
    )(x1, x2, x3, w1, w2, w3, bt)
    return out.T


# speedup vs baseline: 13.2770x; 1.1048x over previous
import jax
import jax.numpy as jnp
from jax.experimental import pallas as pl
from jax.experimental.pallas import tpu as pltpu

_BN = 32000


def _mpn_block_kernel(x1_ref, x2_ref, x3_ref, w1_ref, w2_ref, w3_ref, b_ref,
                      out_ref):
    dn = (((0,), (0,)), ((), ()))
    a1 = jax.lax.dot_general(w1_ref[...], x1_ref[...], dn,
                             preferred_element_type=jnp.float32)
    a2 = jax.lax.dot_general(w2_ref[...], x2_ref[...], dn,
                             preferred_element_type=jnp.float32)
    a3 = jax.lax.dot_general(w3_ref[...], x3_ref[...], dn,
                             preferred_element_type=jnp.float32)
    acc = a1 + a2 + a3 + b_ref[...][:, 0:1]
    out_ref[...] = jnp.maximum(acc, 0.0)


def kernel(f_src, f, sum_msg, w1, w2, w3, b):
    e, d_ndata = f_src.shape
    d_edata = f.shape[1]
    d_msg = sum_msg.shape[1]

    x1 = f_src.T
    x2 = f.T
    x3 = sum_msg.T
    bt = jnp.tile(b.reshape(d_msg, 1), (1, 128))

    bn = _BN if e % _BN == 0 else e
    grid = e // bn

    out = pl.pallas_call(
        _mpn_block_kernel,
        grid=(grid,),
        in_specs=[
            pl.BlockSpec((d_ndata, bn), lambda i: (0, i)),
            pl.BlockSpec((d_edata, bn), lambda i: (0, i)),
            pl.BlockSpec((d_msg, bn), lambda i: (0, i)),
            pl.BlockSpec((d_ndata, d_msg), lambda i: (0, 0)),
            pl.BlockSpec((d_edata, d_msg), lambda i: (0, 0)),
            pl.BlockSpec((d_msg, d_msg), lambda i: (0, 0)),
            pl.BlockSpec((d_msg, 128), lambda i: (0, 0)),
        ],
        out_specs=pl.BlockSpec((d_msg, bn), lambda i: (0, i)),
        out_shape=jax.ShapeDtypeStruct((d_msg, e), jnp.float32),
        compiler_params=pltpu.CompilerParams(
            dimension_semantics=("parallel",)),
    )(x1, x2, x3, w1, w2, w3, bt)
    return out.T


# speedup vs baseline: 13.5336x; 1.0193x over previous
import jax
import jax.numpy as jnp
from jax.experimental import pallas as pl
from jax.experimental.pallas import tpu as pltpu

_CH = 12800
_NBUF = 4


def _body(x1_hbm, x2_hbm, x3_hbm, w1_ref, w2_ref, w3_ref, b_ref, out_hbm,
          b1, b2, b3, bo, in_sem, out_sem):
    e = x1_hbm.shape[1]
    nchunks = e // _CH

    def in_copies(i, slot):
        c = i * _CH
        return (
            pltpu.make_async_copy(x1_hbm.at[:, pl.ds(c, _CH)], b1.at[slot],
                                  in_sem.at[slot, 0]),
            pltpu.make_async_copy(x2_hbm.at[:, pl.ds(c, _CH)], b2.at[slot],
                                  in_sem.at[slot, 1]),
            pltpu.make_async_copy(x3_hbm.at[:, pl.ds(c, _CH)], b3.at[slot],
                                  in_sem.at[slot, 2]),
        )

    def out_copy(i, slot):
        return pltpu.make_async_copy(bo.at[slot],
                                     out_hbm.at[:, pl.ds(i * _CH, _CH)],
                                     out_sem.at[slot])

    for s in range(_NBUF):
        for cp in in_copies(s, s):
            cp.start()

    dn = (((0,), (0,)), ((), ()))

    def step(i, carry):
        slot = jax.lax.rem(i, _NBUF)
        for cp in in_copies(i, slot):
            cp.wait()

        @pl.when(i >= _NBUF)
        def _():
            out_copy(i - _NBUF, slot).wait()

        acc = jax.lax.dot_general(w1_ref[...], b1[slot], dn,
                                  preferred_element_type=jnp.float32)
        acc = acc + jax.lax.dot_general(w2_ref[...], b2[slot], dn,
                                        preferred_element_type=jnp.float32)
        acc = acc + jax.lax.dot_general(w3_ref[...], b3[slot], dn,
                                        preferred_element_type=jnp.float32)
        acc = acc + b_ref[...][:, 0:1]
        bo[slot] = jnp.maximum(acc, 0.0)
        out_copy(i, slot).start()

        @pl.when(i + _NBUF < nchunks)
        def _():
            for cp in in_copies(i + _NBUF, slot):
                cp.start()

        return carry

    jax.lax.fori_loop(0, nchunks, step, 0)

    for s in range(_NBUF):
        i = nchunks - _NBUF + s
        out_copy(i, jax.lax.rem(i, _NBUF)).wait()


def kernel(f_src, f, sum_msg, w1, w2, w3, b):
    e, d_ndata = f_src.shape
    d_edata = f.shape[1]
    d_msg = sum_msg.shape[1]

    x1 = f_src.T
    x2 = f.T
    x3 = sum_msg.T
    bt = jnp.tile(b.reshape(d_msg, 1), (1, 128))

    out = pl.pallas_call(
        _body,
        in_specs=[
            pl.BlockSpec(memory_space=pl.ANY),
            pl.BlockSpec(memory_space=pl.ANY),
            pl.BlockSpec(memory_space=pl.ANY),
            pl.BlockSpec(memory_space=pltpu.MemorySpace.VMEM),
            pl.BlockSpec(memory_space=pltpu.MemorySpace.VMEM),
            pl.BlockSpec(memory_space=pltpu.MemorySpace.VMEM),
            pl.BlockSpec(memory_space=pltpu.MemorySpace.VMEM),
        ],
        out_specs=pl.BlockSpec(memory_space=pl.ANY),
        out_shape=jax.ShapeDtypeStruct((d_msg, e), jnp.float32),
        scratch_shapes=[
            pltpu.VMEM((_NBUF, d_ndata, _CH), jnp.float32),
            pltpu.VMEM((_NBUF, d_edata, _CH), jnp.float32),
            pltpu.VMEM((_NBUF, d_msg, _CH), jnp.float32),
            pltpu.VMEM((_NBUF, d_msg, _CH), jnp.float32),
            pltpu.SemaphoreType.DMA((_NBUF, 3)),
            pltpu.SemaphoreType.DMA((_NBUF,)),
        ],
    )(x1, x2, x3, w1, w2, w3, bt)
    return out.T
